# baseline (speedup 1.0000x reference)
import functools

import jax
import jax.numpy as jnp
from jax import lax
from jax.experimental import pallas as pl
from jax.experimental.pallas import tpu as pltpu
from jax.experimental.pallas import tpu_sc as plsc

_B = 1024
_C = 64
_T = 512
_V = 1000
_BB = 64
_CP = 128

_info = plsc.get_sparse_core_info()
_NW = _info.num_cores * _info.num_subcores
_BPW = _B // _NW

_sc_mesh = plsc.VectorSubcoreMesh(core_axis_name="c", subcore_axis_name="s")


@functools.partial(
    pl.kernel,
    mesh=_sc_mesh,
    out_type=jax.ShapeDtypeStruct((_B, _CP), jnp.float32),
    scratch_types=[
        pltpu.VMEM((_BPW,), jnp.int32),
        pltpu.VMEM((_BPW, _CP), jnp.float32),
        pltpu.SemaphoreType.DMA,
    ],
)
def _sc_gather(idx_hbm, table_hbm, out_hbm, idx_v, rows_v, sem):
    wid = lax.axis_index("s") * _info.num_cores + lax.axis_index("c")
    base = wid * _BPW
    pltpu.sync_copy(idx_hbm.at[pl.ds(base, _BPW)], idx_v)
    pltpu.async_copy(table_hbm.at[idx_v], rows_v, sem).wait()
    pltpu.sync_copy(rows_v, out_hbm.at[pl.ds(base, _BPW)])


def _fused_kernel(emb_ref, wsc_ref, bsc_ref, wsh_ref, bsh_ref,
                  eeg_ref, out_ref):
    emb = emb_ref[...][:, :_C]
    scale = lax.dot_general(emb, wsc_ref[...], (((1,), (1,)), ((), ())),
                            preferred_element_type=jnp.float32) + bsc_ref[...]
    shift = lax.dot_general(emb, wsh_ref[...], (((1,), (1,)), ((), ())),
                            preferred_element_type=jnp.float32) + bsh_ref[...]
    s1 = 1.0 + scale
    for j in range(_BB):
        out_ref[j] = (eeg_ref[j] * s1[j, :, None] + shift[j, :, None])


def kernel(eeg, subject_idx, emb_table, W_scale, b_scale, W_shift, b_shift):
    table_p = jnp.pad(emb_table, ((0, 0), (0, _CP - _C)))
    emb = _sc_gather(subject_idx.astype(jnp.int32), table_p)
    bsc = b_scale.reshape(1, _C)
    bsh = b_shift.reshape(1, _C)

    resident = lambda shape: pl.BlockSpec(shape, lambda i: (0,) * len(shape))
    out = pl.pallas_call(
        _fused_kernel,
        grid=(_B // _BB,),
        in_specs=[
            pl.BlockSpec((_BB, _CP), lambda i: (i, 0)),
            resident((_C, _C)),
            resident((1, _C)),
            resident((_C, _C)),
            resident((1, _C)),
            pl.BlockSpec((_BB, _C, _T), lambda i: (i, 0, 0)),
        ],
        out_specs=pl.BlockSpec((_BB, _C, _T), lambda i: (i, 0, 0)),
        out_shape=jax.ShapeDtypeStruct((_B, _C, _T), jnp.float32),
        compiler_params=pltpu.CompilerParams(
            dimension_semantics=("arbitrary",)),
    )(emb, W_scale, bsc, W_shift, bsh, eeg)
    return out
